# baseline (speedup 1.0000x reference)
import dataclasses

import jax
import jax.numpy as jnp
from jax import lax
from jax.experimental import pallas as pl
from jax.experimental.pallas import tpu as pltpu
from jax.experimental.pallas import tpu_sc as plsc

N = 16384
S_SC = 1024
S_TC = N - S_SC
BM = 256
NW = 32
ROWS_PER_W = S_SC // NW
R = 8
W = 4096
NQ = N // W
NT = (ROWS_PER_W // R) * NQ
LANES = 16


def _tc_body(m_ref, v_ref, r_ref, out_ref, sq_ref, mx_ref):
    i = pl.program_id(0)
    out = jnp.dot(m_ref[...], v_ref[...],
                  preferred_element_type=jnp.float32)[:, 0]
    out_ref[...] = out
    err = out - r_ref[...]
    sq = jnp.sum(err * err).reshape(1, 1)
    mx = jnp.max(jnp.abs(err)).reshape(1, 1)

    @pl.when(i == 0)
    def _init():
        sq_ref[...] = sq
        mx_ref[...] = mx

    @pl.when(i > 0)
    def _acc():
        sq_ref[...] += sq
        mx_ref[...] = jnp.maximum(mx_ref[...], mx)


def _tc_matvec(matrix, vector, ref):
    return pl.pallas_call(
        _tc_body,
        grid=(S_TC // BM,),
        in_specs=[
            pl.BlockSpec((BM, N), lambda i: (i, 0)),
            pl.BlockSpec((N, 1), lambda i: (0, 0)),
            pl.BlockSpec((BM,), lambda i: (i,)),
        ],
        out_specs=[
            pl.BlockSpec((BM,), lambda i: (i,)),
            pl.BlockSpec((1, 1), lambda i: (0, 0)),
            pl.BlockSpec((1, 1), lambda i: (0, 0)),
        ],
        out_shape=[
            jax.ShapeDtypeStruct((S_TC,), jnp.float32),
            jax.ShapeDtypeStruct((1, 1), jnp.float32),
            jax.ShapeDtypeStruct((1, 1), jnp.float32),
        ],
    )(matrix, vector, ref)


def _sc_body(m_hbm, v_hbm, r_hbm, o_hbm, sqp_hbm, mxp_hbm,
             v_vmem, buf0, buf1, tmp, outv, refv, statv, accv,
             sem_v, sem0, sem1, sem_r, sem_o):
    c = lax.axis_index("core")
    s = lax.axis_index("subcore")
    w = s * 2 + c
    base = S_TC + w * ROWS_PER_W

    pltpu.async_copy(v_hbm, v_vmem, sem_v)
    pltpu.async_copy(r_hbm.at[pl.ds(base, ROWS_PER_W)], refv, sem_r)

    def issue(t, buf, sem):
        q = t // NQ
        h = t % NQ
        row0 = base + q * R
        pltpu.async_copy(m_hbm.at[pl.ds(row0, R), pl.ds(h * W, W)], buf, sem)

    def wait_slab(buf, sem):
        pltpu.make_async_copy(m_hbm.at[pl.ds(0, R), pl.ds(0, W)], buf,
                              sem).wait()

    def compute_slab(buf, t):
        q = t // NQ
        h = t % NQ
        voff = h * W
        first = h == 0
        init = [jnp.where(first, 0.0, 1.0) * accv[pl.ds(r * LANES, LANES)]
                for r in range(R)]

        def step(i, carry):
            off = i * LANES
            vc = v_vmem[pl.ds(voff + off, LANES)]
            return tuple(carry[r] + buf[r, pl.ds(off, LANES)] * vc
                         for r in range(R))

        accs = lax.fori_loop(0, W // LANES, step, tuple(init), unroll=4)
        for r in range(R):
            accv[pl.ds(r * LANES, LANES)] = accs[r]

        @pl.when(h == NQ - 1)
        def _():
            row0 = q * R
            for r in range(R):
                tmp[pl.ds((row0 + r) * LANES, LANES)] = accs[r]

    pltpu.make_async_copy(v_hbm, v_vmem, sem_v).wait()
    issue(0, buf0, sem0)

    @pl.loop(0, NT, step=2)
    def _(t):
        issue(t + 1, buf1, sem1)
        wait_slab(buf0, sem0)
        compute_slab(buf0, t)

        @pl.when(t + 2 < NT)
        def _():
            issue(t + 2, buf0, sem0)

        wait_slab(buf1, sem1)
        compute_slab(buf1, t + 1)

    pltpu.make_async_copy(r_hbm.at[pl.ds(base, ROWS_PER_W)], refv, sem_r).wait()
    rows16 = lax.iota(jnp.int32, LANES)

    def group_step(g, carry):
        sq, mx = carry
        ridx = (rows16 + g * LANES) * LANES
        acc = jnp.zeros((LANES,), jnp.float32)
        for l in range(LANES):
            acc = acc + plsc.load_gather(tmp, [ridx + l])
        sl = pl.ds(g * LANES, LANES)
        outv[sl] = acc
        e = acc - refv[sl]
        return (sq + e * e, jnp.maximum(mx, jnp.abs(e)))

    z = jnp.zeros((LANES,), jnp.float32)
    sq, mx = lax.fori_loop(0, ROWS_PER_W // LANES, group_step, (z, z))
    obase = w * ROWS_PER_W
    pltpu.async_copy(outv, o_hbm.at[pl.ds(obase, ROWS_PER_W)], sem_o)
    statv[pl.ds(0, LANES)] = sq
    statv[pl.ds(LANES, LANES)] = mx
    pltpu.sync_copy(statv.at[pl.ds(0, LANES)], sqp_hbm.at[pl.ds(w * LANES, LANES)])
    pltpu.sync_copy(statv.at[pl.ds(LANES, LANES)],
                    mxp_hbm.at[pl.ds(w * LANES, LANES)])
    pltpu.make_async_copy(outv, o_hbm.at[pl.ds(obase, ROWS_PER_W)], sem_o).wait()


def _sc_matvec(mflat, vec1d, ref):
    mesh = plsc.VectorSubcoreMesh(core_axis_name="core",
                                  subcore_axis_name="subcore")
    cp = pltpu.CompilerParams()
    if "needs_layout_passes" in pltpu.CompilerParams.__dataclass_fields__:
        cp = dataclasses.replace(cp, needs_layout_passes=False)
    f = pl.kernel(
        _sc_body,
        compiler_params=cp,
        out_type=[
            jax.ShapeDtypeStruct((S_SC,), jnp.float32),
            jax.ShapeDtypeStruct((NW * LANES,), jnp.float32),
            jax.ShapeDtypeStruct((NW * LANES,), jnp.float32),
        ],
        mesh=mesh,
        scratch_types=[
            pltpu.VMEM((N,), jnp.float32),
            pltpu.VMEM((R, W), jnp.float32),
            pltpu.VMEM((R, W), jnp.float32),
            pltpu.VMEM((ROWS_PER_W * LANES,), jnp.float32),
            pltpu.VMEM((ROWS_PER_W,), jnp.float32),
            pltpu.VMEM((ROWS_PER_W,), jnp.float32),
            pltpu.VMEM((2 * LANES,), jnp.float32),
            pltpu.VMEM((R * LANES,), jnp.float32),
            pltpu.SemaphoreType.DMA,
            pltpu.SemaphoreType.DMA,
            pltpu.SemaphoreType.DMA,
            pltpu.SemaphoreType.DMA,
            pltpu.SemaphoreType.DMA,
        ],
    )
    return f(mflat, vec1d, ref)


def kernel(matrix, vector, ref):
    out_sc, sqp, mxp = _sc_matvec(matrix, vector.reshape(N), ref)
    out_tc, sq_tc, mx_tc = _tc_matvec(matrix, vector, ref)
    out = jnp.concatenate([out_tc, out_sc])
    loss = (sq_tc[0, 0] + jnp.sum(sqp)) / jnp.float32(N)
    mx = jnp.maximum(mx_tc[0, 0], jnp.max(mxp))
    return (loss, out, ref, mx)


# speedup vs baseline: 1.0462x; 1.0462x over previous
import jax
import jax.numpy as jnp
from jax.experimental import pallas as pl

N = 16384
BM = 128
NBLK = N // BM


def _body(m_ref, v_ref, r_ref, out_ref, sq_ref, mx_ref):
    i = pl.program_id(0)
    out = jnp.dot(m_ref[...], v_ref[...],
                  preferred_element_type=jnp.float32)[:, 0]
    out_ref[...] = out
    err = out - r_ref[...]
    sq = jnp.sum(err * err).reshape(1, 1)
    mx = jnp.max(jnp.abs(err)).reshape(1, 1)

    @pl.when(i == 0)
    def _init():
        sq_ref[...] = sq
        mx_ref[...] = mx

    @pl.when(i > 0)
    def _acc():
        sq_ref[...] += sq
        mx_ref[...] = jnp.maximum(mx_ref[...], mx)


def kernel(matrix, vector, ref):
    out, sq, mx = pl.pallas_call(
        _body,
        grid=(NBLK,),
        in_specs=[
            pl.BlockSpec((BM, N), lambda i: (i, 0)),
            pl.BlockSpec((N, 1), lambda i: (0, 0)),
            pl.BlockSpec((BM,), lambda i: (i,)),
        ],
        out_specs=[
            pl.BlockSpec((BM,), lambda i: (i,)),
            pl.BlockSpec((1, 1), lambda i: (0, 0)),
            pl.BlockSpec((1, 1), lambda i: (0, 0)),
        ],
        out_shape=[
            jax.ShapeDtypeStruct((N,), jnp.float32),
            jax.ShapeDtypeStruct((1, 1), jnp.float32),
            jax.ShapeDtypeStruct((1, 1), jnp.float32),
        ],
    )(matrix, vector, ref)
    loss = sq[0, 0] / jnp.float32(N)
    return (loss, out, ref, mx[0, 0])


# speedup vs baseline: 1.0820x; 1.0342x over previous
import jax
import jax.numpy as jnp
from jax.experimental import pallas as pl

N = 16384
BM = 256
NBLK = N // BM


def _body(m_ref, v_ref, r_ref, out_ref, sq_ref, mx_ref):
    i = pl.program_id(0)
    out = jnp.dot(m_ref[...], v_ref[...],
                  preferred_element_type=jnp.float32)[:, 0]
    out_ref[...] = out
    err = out - r_ref[...]
    sq = jnp.sum(err * err).reshape(1, 1)
    mx = jnp.max(jnp.abs(err)).reshape(1, 1)

    @pl.when(i == 0)
    def _init():
        sq_ref[...] = sq
        mx_ref[...] = mx

    @pl.when(i > 0)
    def _acc():
        sq_ref[...] += sq
        mx_ref[...] = jnp.maximum(mx_ref[...], mx)


def kernel(matrix, vector, ref):
    out, sq, mx = pl.pallas_call(
        _body,
        grid=(NBLK,),
        in_specs=[
            pl.BlockSpec((BM, N), lambda i: (i, 0)),
            pl.BlockSpec((N, 1), lambda i: (0, 0)),
            pl.BlockSpec((BM,), lambda i: (i,)),
        ],
        out_specs=[
            pl.BlockSpec((BM,), lambda i: (i,)),
            pl.BlockSpec((1, 1), lambda i: (0, 0)),
            pl.BlockSpec((1, 1), lambda i: (0, 0)),
        ],
        out_shape=[
            jax.ShapeDtypeStruct((N,), jnp.float32),
            jax.ShapeDtypeStruct((1, 1), jnp.float32),
            jax.ShapeDtypeStruct((1, 1), jnp.float32),
        ],
    )(matrix, vector, ref)
    loss = sq[0, 0] / jnp.float32(N)
    return (loss, out, ref, mx[0, 0])


# speedup vs baseline: 1.1296x; 1.0440x over previous
import jax
import jax.numpy as jnp
from jax.experimental import pallas as pl

N = 16384
BM = 256
NBLK = N // BM


def _body(m_ref, v_ref, r_ref, out_ref, sq_ref, mx_ref):
    i = pl.program_id(0)
    out = jax.lax.dot_general(
        m_ref[...], v_ref[...],
        dimension_numbers=(((1,), (1,)), ((), ())),
        preferred_element_type=jnp.float32)[:, 0]
    out_ref[...] = out
    err = out - r_ref[...]
    sq = jnp.sum(err * err).reshape(1, 1)
    mx = jnp.max(jnp.abs(err)).reshape(1, 1)

    @pl.when(i == 0)
    def _init():
        sq_ref[...] = sq
        mx_ref[...] = mx

    @pl.when(i > 0)
    def _acc():
        sq_ref[...] += sq
        mx_ref[...] = jnp.maximum(mx_ref[...], mx)


def kernel(matrix, vector, ref):
    out, sq, mx = pl.pallas_call(
        _body,
        grid=(NBLK,),
        in_specs=[
            pl.BlockSpec((BM, N), lambda i: (i, 0)),
            pl.BlockSpec((1, N), lambda i: (0, 0)),
            pl.BlockSpec((BM,), lambda i: (i,)),
        ],
        out_specs=[
            pl.BlockSpec((BM,), lambda i: (i,)),
            pl.BlockSpec((1, 1), lambda i: (0, 0)),
            pl.BlockSpec((1, 1), lambda i: (0, 0)),
        ],
        out_shape=[
            jax.ShapeDtypeStruct((N,), jnp.float32),
            jax.ShapeDtypeStruct((1, 1), jnp.float32),
            jax.ShapeDtypeStruct((1, 1), jnp.float32),
        ],
    )(matrix, vector.reshape(1, N), ref)
    loss = sq[0, 0] / jnp.float32(N)
    return (loss, out, ref, mx[0, 0])
